# Initial kernel scaffold; baseline (speedup 1.0000x reference)
import jax
import jax.numpy as jnp
from jax.experimental import pallas as pl


def kernel(user, question, response, mask, q_neighbors, s_neighbors, u_neighbors, q_neighbors_2, qs_table, emb_q, emb_s, emb_u, emb_q2, emb_r, W_ih, W_hh, b_ih, b_hh, agg_W, agg_b, W_agg_last, b_agg_last, W_att, b_att, W_query, b_query, W_key, b_key, W_w, b_w):
    raise NotImplementedError("write your pallas kernel here")


# speedup vs baseline: 5.9617x; 5.9617x over previous
import functools
import jax
import jax.numpy as jnp
from jax import lax
from jax.experimental import pallas as pl
from jax.experimental.pallas import tpu as pltpu
from jax.experimental.pallas import tpu_sc as plsc

NQ, NS, NU = 20000, 1000, 5000
D = 100
DP = 128
NBR = 8
SPQ = 4
B, S = 64, 64
N = B * S
NSP = 1024
NQP = 20480

NC, NSC = 2, 16
NW = NC * NSC
CH = N // NW
MS_CH = NSP // NW
MU_CH = NQP // NW


def _wid():
    return lax.axis_index("s") * NC + lax.axis_index("c")


def _acc_rows(acc, buf, nrows):
    def body(r, _):
        for c in range(DP // 16):
            sl = pl.ds(c * 16, 16)
            acc[r, sl] = acc[r, sl] + buf[r, sl]
        return 0
    lax.fori_loop(0, nrows, body, 0)


def _sc_a(qt, ut, qn, qnb, unb, qst, embq, embq2, embu,
          out_nbq, out_nbu, out_qsn, out_gq0, out_gq2sel, out_gu0, out_gqn,
          idx_v, ibuf, fbuf, sem):
    base = _wid() * CH
    sl = pl.ds(base, CH)
    pltpu.sync_copy(qt.at[sl], idx_v)
    pltpu.async_copy(qnb.at[idx_v], ibuf, sem).wait()
    pltpu.sync_copy(ibuf, out_nbq.at[sl])
    pltpu.async_copy(embq.at[idx_v], fbuf, sem).wait()
    pltpu.sync_copy(fbuf, out_gq0.at[sl])
    pltpu.async_copy(embq2.at[idx_v], fbuf, sem).wait()
    pltpu.sync_copy(fbuf, out_gq2sel.at[sl])
    pltpu.sync_copy(ut.at[sl], idx_v)
    pltpu.async_copy(unb.at[idx_v], ibuf, sem).wait()
    pltpu.sync_copy(ibuf, out_nbu.at[sl])
    pltpu.async_copy(embu.at[idx_v], fbuf, sem).wait()
    pltpu.sync_copy(fbuf, out_gu0.at[sl])
    pltpu.sync_copy(qn.at[sl], idx_v)
    pltpu.async_copy(qst.at[idx_v], ibuf, sem).wait()
    pltpu.sync_copy(ibuf, out_qsn.at[sl])
    pltpu.async_copy(embq.at[idx_v], fbuf, sem).wait()
    pltpu.sync_copy(fbuf, out_gqn.at[sl])


def _sc_b(snbT, qn2T, nbqT, nbuT, qsnT, embq, embu, embs, embq2,
          out_ms, out_mu, out_m1, out_mu1, out_skl,
          idx_v, sidx_v, buf, acc, sbuf, sem):
    wid = _wid()

    sbase = wid * MS_CH
    for j in range(NBR):
        pltpu.sync_copy(snbT.at[pl.ds(j * NSP + sbase, MS_CH)], sidx_v)
        dst = acc if j == 0 else buf
        pltpu.async_copy(embq.at[sidx_v], dst.at[pl.ds(0, MS_CH)], sem).wait()
        if j > 0:
            _acc_rows(acc, buf, MS_CH)
    pltpu.sync_copy(acc.at[pl.ds(0, MS_CH)], out_ms.at[pl.ds(sbase, MS_CH)])

    for c in range(MU_CH // CH):
        mbase = wid * MU_CH + c * CH
        for j in range(NBR):
            pltpu.sync_copy(qn2T.at[pl.ds(j * NQP + mbase, CH)], idx_v)
            dst = acc if j == 0 else buf
            pltpu.async_copy(embu.at[idx_v], dst, sem).wait()
            if j > 0:
                _acc_rows(acc, buf, CH)
        pltpu.sync_copy(acc, out_mu.at[pl.ds(mbase, CH)])

    base = wid * CH
    for j in range(NBR):
        pltpu.sync_copy(nbqT.at[pl.ds(j * N + base, CH)], idx_v)
        dst = acc if j == 0 else buf
        pltpu.async_copy(embs.at[idx_v], dst, sem).wait()
        if j > 0:
            _acc_rows(acc, buf, CH)
    pltpu.sync_copy(acc, out_m1.at[pl.ds(base, CH)])

    for j in range(NBR):
        pltpu.sync_copy(nbuT.at[pl.ds(j * N + base, CH)], idx_v)
        dst = acc if j == 0 else buf
        pltpu.async_copy(embq2.at[idx_v], dst, sem).wait()
        if j > 0:
            _acc_rows(acc, buf, CH)
    pltpu.sync_copy(acc, out_mu1.at[pl.ds(base, CH)])

    for j in range(SPQ):
        pltpu.sync_copy(qsnT.at[pl.ds(j * N + base, CH)], idx_v)
        pltpu.async_copy(embs.at[idx_v], sbuf, sem).wait()
        pltpu.sync_copy(sbuf, out_skl.at[pl.ds(j * N + base, CH)])


def _sc_c(nbqT, nbuT, e1, e1u, out_ae1, out_ae1u, idx_v, buf, acc, sem):
    base = _wid() * CH
    for j in range(NBR):
        pltpu.sync_copy(nbqT.at[pl.ds(j * N + base, CH)], idx_v)
        dst = acc if j == 0 else buf
        pltpu.async_copy(e1.at[idx_v], dst, sem).wait()
        if j > 0:
            _acc_rows(acc, buf, CH)
    pltpu.sync_copy(acc, out_ae1.at[pl.ds(base, CH)])
    for j in range(NBR):
        pltpu.sync_copy(nbuT.at[pl.ds(j * N + base, CH)], idx_v)
        dst = acc if j == 0 else buf
        pltpu.async_copy(e1u.at[idx_v], dst, sem).wait()
        if j > 0:
            _acc_rows(acc, buf, CH)
    pltpu.sync_copy(acc, out_ae1u.at[pl.ds(base, CH)])


_f32 = jnp.float32
_i32 = jnp.int32


@functools.cache
def _sc_mesh():
    return plsc.VectorSubcoreMesh(core_axis_name="c", subcore_axis_name="s")


def _run_sc_a(qt, ut, qn, qnb, unb, qst, embq, embq2, embu):
    out_type = [
        jax.ShapeDtypeStruct((N, 128), _i32),
        jax.ShapeDtypeStruct((N, 128), _i32),
        jax.ShapeDtypeStruct((N, 128), _i32),
        jax.ShapeDtypeStruct((N, DP), _f32),
        jax.ShapeDtypeStruct((N, DP), _f32),
        jax.ShapeDtypeStruct((N, DP), _f32),
        jax.ShapeDtypeStruct((N, DP), _f32),
    ]
    scratch = [
        pltpu.VMEM((CH,), _i32),
        pltpu.VMEM((CH, 128), _i32),
        pltpu.VMEM((CH, DP), _f32),
        pltpu.SemaphoreType.DMA,
    ]
    return pl.kernel(_sc_a, out_type=out_type, mesh=_sc_mesh(),
                     scratch_types=scratch)(qt, ut, qn, qnb, unb, qst,
                                            embq, embq2, embu)


def _run_sc_b(snbT, qn2T, nbqT, nbuT, qsnT, embq, embu, embs, embq2):
    out_type = [
        jax.ShapeDtypeStruct((NSP, DP), _f32),
        jax.ShapeDtypeStruct((NQP, DP), _f32),
        jax.ShapeDtypeStruct((N, DP), _f32),
        jax.ShapeDtypeStruct((N, DP), _f32),
        jax.ShapeDtypeStruct((SPQ * N, DP), _f32),
    ]
    scratch = [
        pltpu.VMEM((CH,), _i32),
        pltpu.VMEM((MS_CH,), _i32),
        pltpu.VMEM((CH, DP), _f32),
        pltpu.VMEM((CH, DP), _f32),
        pltpu.VMEM((CH, DP), _f32),
        pltpu.SemaphoreType.DMA,
    ]
    return pl.kernel(_sc_b, out_type=out_type, mesh=_sc_mesh(),
                     scratch_types=scratch)(snbT, qn2T, nbqT, nbuT, qsnT,
                                            embq, embu, embs, embq2)


def _run_sc_c(nbqT, nbuT, e1, e1u):
    out_type = [
        jax.ShapeDtypeStruct((N, DP), _f32),
        jax.ShapeDtypeStruct((N, DP), _f32),
    ]
    scratch = [
        pltpu.VMEM((CH,), _i32),
        pltpu.VMEM((CH, DP), _f32),
        pltpu.VMEM((CH, DP), _f32),
        pltpu.SemaphoreType.DMA,
    ]
    return pl.kernel(_sc_c, out_type=out_type, mesh=_sc_mesh(),
                     scratch_types=scratch)(nbqT, nbuT, e1, e1u)


def _tc_e1(emb_ref, sum_ref, w_ref, b_ref, out_ref):
    x = emb_ref[...] + 0.125 * sum_ref[...]
    out_ref[...] = jnp.tanh(
        jnp.dot(x, w_ref[...], preferred_element_type=_f32) + b_ref[0:1, :])


def _run_tc_e1(emb_all, sum_all, w1p, b1p):
    rows = emb_all.shape[0]
    blk = 512
    return pl.pallas_call(
        _tc_e1,
        grid=(rows // blk,),
        in_specs=[
            pl.BlockSpec((blk, DP), lambda i: (i, 0)),
            pl.BlockSpec((blk, DP), lambda i: (i, 0)),
            pl.BlockSpec((DP, DP), lambda i: (0, 0)),
            pl.BlockSpec((8, DP), lambda i: (0, 0)),
        ],
        out_specs=pl.BlockSpec((blk, DP), lambda i: (i, 0)),
        out_shape=jax.ShapeDtypeStruct((rows, DP), _f32),
    )(emb_all, sum_all, w1p, b1p)


def _tc_rows(gq0, m1s, ae1s, gu0, mu1s, ae1us, gq2sel, mcol, rcol, embr,
             w0, b0, wl, bl, watt, batt, wih, bg, wk, bk, out_k):
    b0r = b0[0:1, :]
    blr = bl[0:1, :]

    x = gq0[...] + 0.125 * m1s[...]
    e0a = jnp.tanh(jnp.dot(x, w0[...], preferred_element_type=_f32) + b0r)
    x = e0a + 0.125 * ae1s[...]
    e0b = jnp.tanh(jnp.dot(x, w0[...], preferred_element_type=_f32) + b0r)
    aggq = jnp.tanh(jnp.dot(e0b, wl[...], preferred_element_type=_f32) + blr)

    x = gu0[...] + 0.125 * mu1s[...]
    e0a = jnp.tanh(jnp.dot(x, w0[...], preferred_element_type=_f32) + b0r)
    x = e0a + 0.125 * ae1us[...]
    e0b = jnp.tanh(jnp.dot(x, w0[...], preferred_element_type=_f32) + b0r)
    aggu = jnp.tanh(jnp.dot(e0b, wl[...], preferred_element_type=_f32) + blr)

    m = mcol[...] > 0.5
    eq1 = jnp.where(m, aggq, gq0[...])
    eq2 = jnp.where(m, aggu, gq2sel[...])
    eq = jnp.concatenate([eq1, eq2], axis=1)
    logits = jnp.dot(eq, watt[...], preferred_element_type=_f32) + batt[0:1, :]
    lmax = jnp.max(logits, axis=1, keepdims=True)
    ew = jnp.exp(logits - lmax)
    attn = ew / jnp.sum(ew, axis=1, keepdims=True)
    eq = eq * attn

    r = rcol[...]
    embr_row = embr[0:1, :] * (1.0 - r) + embr[1:2, :] * r
    x2 = jnp.concatenate([eq, embr_row], axis=1)
    gates = jnp.dot(x2, wih[...], preferred_element_type=_f32) + bg[0:1, :]
    gi = jax.nn.sigmoid(gates[:, 0:DP])
    gg = jnp.tanh(gates[:, 2 * DP:3 * DP])
    go = jax.nn.sigmoid(gates[:, 3 * DP:4 * DP])
    h = go * jnp.tanh(gi * gg)
    out_k[...] = jnp.dot(h, wk[...], preferred_element_type=_f32) + bk[0:1, :]


def _run_tc_rows(gq0, m1s, ae1s, gu0, mu1s, ae1us, gq2sel, mcol, rcol, embr,
                 w0, b0, wl, bl, watt, batt, wih, bg, wk, bk):
    blk = 512
    row_spec = pl.BlockSpec((blk, DP), lambda i: (i, 0))
    col_spec = pl.BlockSpec((blk, 1), lambda i: (i, 0))
    full = lambda a: pl.BlockSpec(a.shape, lambda i: tuple(0 for _ in a.shape))
    return pl.pallas_call(
        _tc_rows,
        grid=(N // blk,),
        in_specs=[row_spec] * 7 + [col_spec, col_spec] +
                 [full(a) for a in (embr, w0, b0, wl, bl, watt, batt, wih, bg,
                                    wk, bk)],
        out_specs=row_spec,
        out_shape=jax.ShapeDtypeStruct((N, DP), _f32),
    )(gq0, m1s, ae1s, gu0, mu1s, ae1us, gq2sel, mcol, rcol, embr,
      w0, b0, wl, bl, watt, batt, wih, bg, wk, bk)


def _tc_pred(qs_ref, k_ref, wq, bq, wwq, wwk, out_ref):
    qs = qs_ref[0]
    k = k_ref[0]
    q = jnp.dot(qs, wq[...], preferred_element_type=_f32) + bq[0:1, :]
    s = lax.dot_general(q, k, (((1,), (1,)), ((), ())),
                        preferred_element_type=_f32)
    qw = jnp.sum(q * wwq[0:1, :], axis=1, keepdims=True)
    kwf = lax.dot_general(wwk[...], k, (((1,), (1,)), ((), ())),
                          preferred_element_type=_f32)
    w = qw + kwf[0:1, :]
    ti = lax.broadcasted_iota(_i32, (S * 5, S), 0) // 5
    jj = lax.broadcasted_iota(_i32, (S * 5, S), 1)
    valid = (jj <= ti) & (ti <= S - 2)
    w = jnp.where(valid, w, -1e30)
    wmax = jnp.max(w)
    ew = jnp.where(valid, jnp.exp(w - wmax), 0.0)
    num = ew * jax.nn.sigmoid(s)
    seg = (lax.broadcasted_iota(_i32, (S, S * 5), 1) // 5
           == lax.broadcasted_iota(_i32, (S, S * 5), 0)).astype(_f32)
    ew_t = jnp.sum(jnp.dot(seg, ew, preferred_element_type=_f32),
                   axis=1, keepdims=True)
    num_t = jnp.sum(jnp.dot(seg, num, preferred_element_type=_f32),
                    axis=1, keepdims=True)
    out_ref[...] = (num_t / (ew_t + 1e-30)).reshape(1, S, 1)


def _run_tc_pred(qs_all, k_all, wqp, bqp, wwq, wwk):
    return pl.pallas_call(
        _tc_pred,
        grid=(B,),
        in_specs=[
            pl.BlockSpec((1, S * 5, DP), lambda b: (b, 0, 0)),
            pl.BlockSpec((1, S, DP), lambda b: (b, 0, 0)),
            pl.BlockSpec((DP, DP), lambda b: (0, 0)),
            pl.BlockSpec((8, DP), lambda b: (0, 0)),
            pl.BlockSpec((8, DP), lambda b: (0, 0)),
            pl.BlockSpec((8, DP), lambda b: (0, 0)),
        ],
        out_specs=pl.BlockSpec((1, S, 1), lambda b: (b, 0, 0)),
        out_shape=jax.ShapeDtypeStruct((B, S, 1), _f32),
    )(qs_all, k_all, wqp, bqp, wwq, wwk)


def _pad_rows_cols(a, rows, cols):
    return jnp.pad(a, ((0, rows - a.shape[0]), (0, cols - a.shape[1])))


def _pad_vec_row(v, cols, fill=0.0):
    vp = jnp.pad(v.astype(_f32), (0, cols - v.shape[0]),
                 constant_values=fill)
    return jnp.broadcast_to(vp[None, :], (8, cols))


def _pad_block_matrix(w, in_blocks, out_blocks, blk_in=D, blk_out=D):
    out = jnp.zeros((in_blocks * DP, out_blocks * DP), _f32)
    for i in range(in_blocks):
        for j in range(out_blocks):
            out = out.at[i * DP:i * DP + blk_in, j * DP:j * DP + blk_out].set(
                w[i * blk_in:(i + 1) * blk_in, j * blk_out:(j + 1) * blk_out])
    return out


def _pad_block_vec(v, blocks, fill=0.0):
    out = jnp.full((blocks * DP,), fill, _f32)
    for i in range(blocks):
        out = out.at[i * DP:i * DP + D].set(v[i * D:(i + 1) * D])
    return jnp.broadcast_to(out[None, :], (8, blocks * DP))


def kernel(user, question, response, mask, q_neighbors, s_neighbors,
           u_neighbors, q_neighbors_2, qs_table, emb_q, emb_s, emb_u,
           emb_q2, emb_r, W_ih, W_hh, b_ih, b_hh, agg_W, agg_b,
           W_agg_last, b_agg_last, W_att, b_att, W_query, b_query,
           W_key, b_key, W_w, b_w):
    i32 = lambda a: a.astype(_i32)
    qt = i32(question.reshape(-1))
    ut = i32(user.reshape(-1))
    qn = i32(jnp.concatenate([question[:, 1:], question[:, -1:]],
                             axis=1).reshape(-1))

    embq_p = jnp.pad(emb_q, ((0, 0), (0, DP - D)))
    embs_p = jnp.pad(emb_s, ((0, 0), (0, DP - D)))
    embu_p = jnp.pad(emb_u, ((0, 0), (0, DP - D)))
    embq2_p = jnp.pad(emb_q2, ((0, 0), (0, DP - D)))
    qnb_p = jnp.pad(i32(q_neighbors), ((0, 0), (0, 128 - NBR)))
    unb_p = jnp.pad(i32(u_neighbors), ((0, 0), (0, 128 - NBR)))
    qst_p = jnp.pad(i32(qs_table), ((0, 0), (0, 128 - SPQ)))
    snbT = jnp.pad(i32(s_neighbors), ((0, NSP - NS), (0, 0))).T.reshape(-1)
    qn2T = jnp.pad(i32(q_neighbors_2), ((0, NQP - NQ), (0, 0))).T.reshape(-1)

    nbq, nbu, qsn, g_q0, g_q2sel, g_u0, g_qnext = _run_sc_a(
        qt, ut, qn, qnb_p, unb_p, qst_p, embq_p, embq2_p, embu_p)
    nbqT = nbq[:, :NBR].T.reshape(-1)
    nbuT = nbu[:, :NBR].T.reshape(-1)
    qsnT = qsn[:, :SPQ].T.reshape(-1)

    ms_sum, mu_sum, m1_sum, mu1_sum, skl = _run_sc_b(
        snbT, qn2T, nbqT, nbuT, qsnT, embq_p, embu_p, embs_p, embq2_p)

    w1p = _pad_rows_cols(agg_W[1], DP, DP)
    b1p = _pad_vec_row(agg_b[1], DP)
    emb_tab = jnp.concatenate(
        [jnp.pad(embs_p, ((0, NSP - NS), (0, 0))),
         jnp.pad(embq2_p, ((0, NQP - NQ), (0, 0)))], axis=0)
    sum_tab = jnp.concatenate([ms_sum, mu_sum], axis=0)
    e_all = _run_tc_e1(emb_tab, sum_tab, w1p, b1p)
    e1 = e_all[:NSP]
    e1u = e_all[NSP:]

    ae1_sum, ae1u_sum = _run_sc_c(nbqT, nbuT, e1, e1u)

    w0p = _pad_rows_cols(agg_W[0], DP, DP)
    b0p = _pad_vec_row(agg_b[0], DP)
    wlp = _pad_rows_cols(W_agg_last, DP, DP)
    blp = _pad_vec_row(b_agg_last, DP)
    wattp = _pad_block_matrix(W_att, 2, 2)
    battp = _pad_block_vec(b_att, 2, fill=-1e30)
    wihp = _pad_block_matrix(W_ih.T, 3, 4)
    bgp = _pad_block_vec(b_ih + b_hh, 4)
    wkp = _pad_rows_cols(W_key, DP, DP)
    bkp = _pad_vec_row(b_key, DP)
    embr_p = jnp.pad(emb_r, ((0, 6), (0, DP - D)))
    mcol = (mask.reshape(-1, 1)).astype(_f32)
    rcol = (response.reshape(-1, 1)).astype(_f32)
    k_all = _run_tc_rows(g_q0, m1_sum, ae1_sum, g_u0, mu1_sum, ae1u_sum,
                         g_q2sel, mcol, rcol, embr_p,
                         w0p, b0p, wlp, blp, wattp, battp, wihp, bgp,
                         wkp, bkp)

    qs_stack = jnp.stack(
        [g_qnext] + [skl[j * N:(j + 1) * N] for j in range(SPQ)],
        axis=1)
    qs_all = qs_stack.reshape(B, S * 5, DP)
    k_resh = k_all.reshape(B, S, DP)
    wqp = _pad_rows_cols(W_query, DP, DP)
    bqp = _pad_vec_row(b_query, DP)
    wwq = _pad_vec_row(W_w[:D, 0], DP)
    wwk = _pad_vec_row(W_w[D:, 0], DP)
    out_T = _run_tc_pred(qs_all, k_resh, wqp, bqp, wwq, wwk)

    y = out_T[:, :, 0]
    return jnp.concatenate([jnp.full((B, 1), 0.5, _f32), y[:, :S - 1]],
                           axis=1)


# speedup vs baseline: 6.8137x; 1.1429x over previous
import functools
import jax
import jax.numpy as jnp
from jax import lax
from jax.experimental import pallas as pl
from jax.experimental.pallas import tpu as pltpu
from jax.experimental.pallas import tpu_sc as plsc

NQ, NS, NU = 20000, 1000, 5000
D = 100
DP = 128
NBR = 8
SPQ = 4
B, S = 64, 64
N = B * S
NSP = 1024
NQP = 20480

NC, NSC = 2, 16
NW = NC * NSC
CH = N // NW
MS_CH = NSP // NW
MU_CH = NQP // NW


def _wid():
    return lax.axis_index("s") * NC + lax.axis_index("c")


def _acc(acc, buf, nrows):
    def body(r, _):
        for c in range(7):
            sl = pl.ds(c * 16, 16)
            plsc.addupdate(acc.at[r, sl], buf[r, sl])
        return 0
    lax.fori_loop(0, nrows, body, 0, unroll=2)


def _gsum8(tab, idx2d, joff, nrows, acc, b0, b1, s0, s1):
    dst_a = acc if nrows == CH else acc.at[pl.ds(0, nrows)]
    dst0 = b0 if nrows == CH else b0.at[pl.ds(0, nrows)]
    dst1 = b1 if nrows == CH else b1.at[pl.ds(0, nrows)]
    cps = [None] * NBR
    cps[0] = pltpu.async_copy(tab.at[idx2d.at[joff]], dst_a, s0)
    cps[1] = pltpu.async_copy(tab.at[idx2d.at[joff + 1]], dst1, s1)
    cps[0].wait()
    for j in range(2, NBR + 1):
        if j < NBR:
            cps[j] = pltpu.async_copy(
                tab.at[idx2d.at[joff + j]],
                dst0 if j % 2 == 0 else dst1,
                s0 if j % 2 == 0 else s1)
        cps[j - 1].wait()
        _acc(acc, b0 if (j - 1) % 2 == 0 else b1, nrows)


def _sc_a(qt, ut, qn, qnb, unb, qst, embq, embq2, embu,
          out_nbq, out_nbu, out_qsn, out_gq0, out_gq2sel, out_gu0, out_gqn,
          idx_v, ibuf, fbuf, sem):
    base = _wid() * CH
    sl = pl.ds(base, CH)
    pltpu.sync_copy(qt.at[sl], idx_v)
    pltpu.async_copy(qnb.at[idx_v], ibuf, sem).wait()
    pltpu.sync_copy(ibuf, out_nbq.at[sl])
    pltpu.async_copy(embq.at[idx_v], fbuf, sem).wait()
    pltpu.sync_copy(fbuf, out_gq0.at[sl])
    pltpu.async_copy(embq2.at[idx_v], fbuf, sem).wait()
    pltpu.sync_copy(fbuf, out_gq2sel.at[sl])
    pltpu.sync_copy(ut.at[sl], idx_v)
    pltpu.async_copy(unb.at[idx_v], ibuf, sem).wait()
    pltpu.sync_copy(ibuf, out_nbu.at[sl])
    pltpu.async_copy(embu.at[idx_v], fbuf, sem).wait()
    pltpu.sync_copy(fbuf, out_gu0.at[sl])
    pltpu.sync_copy(qn.at[sl], idx_v)
    pltpu.async_copy(qst.at[idx_v], ibuf, sem).wait()
    pltpu.sync_copy(ibuf, out_qsn.at[sl])
    pltpu.async_copy(embq.at[idx_v], fbuf, sem).wait()
    pltpu.sync_copy(fbuf, out_gqn.at[sl])


def _sc_b(mu_img, ms_img, nbq_img, nbu_img, qsn_img, embq, embu, embs, embq2,
          out_ms, out_mu, out_m1, out_mu1, out_skl,
          idxm, idxs_, idx8a, idx8b, idx4, acc, b0, b1, s0, s1):
    wid = _wid()
    pltpu.sync_copy(mu_img.at[wid], idxm)
    pltpu.sync_copy(ms_img.at[wid], idxs_)
    pltpu.sync_copy(nbq_img.at[wid], idx8a)
    pltpu.sync_copy(nbu_img.at[wid], idx8b)
    pltpu.sync_copy(qsn_img.at[wid], idx4)

    sbase = wid * MS_CH
    _gsum8(embq, idxs_, 0, MS_CH, acc, b0, b1, s0, s1)
    pltpu.sync_copy(acc.at[pl.ds(0, MS_CH)], out_ms.at[pl.ds(sbase, MS_CH)])

    for c in range(MU_CH // CH):
        _gsum8(embu, idxm, c * NBR, CH, acc, b0, b1, s0, s1)
        pltpu.sync_copy(acc, out_mu.at[pl.ds(wid * MU_CH + c * CH, CH)])

    base = wid * CH
    _gsum8(embs, idx8a, 0, CH, acc, b0, b1, s0, s1)
    pltpu.sync_copy(acc, out_m1.at[pl.ds(base, CH)])

    _gsum8(embq2, idx8b, 0, CH, acc, b0, b1, s0, s1)
    pltpu.sync_copy(acc, out_mu1.at[pl.ds(base, CH)])

    cps = [None] * SPQ
    cps[0] = pltpu.async_copy(embs.at[idx4.at[0]], b0, s0)
    cps[1] = pltpu.async_copy(embs.at[idx4.at[1]], b1, s1)
    for j in range(SPQ):
        cps[j].wait()
        pltpu.sync_copy(b0 if j % 2 == 0 else b1,
                        out_skl.at[pl.ds(j * N + base, CH)])
        if j + 2 < SPQ:
            cps[j + 2] = pltpu.async_copy(
                embs.at[idx4.at[j + 2]],
                b0 if (j + 2) % 2 == 0 else b1,
                s0 if (j + 2) % 2 == 0 else s1)


def _sc_c(nbq_img, nbu_img, e1, e1u, out_ae1, out_ae1u,
          idx8a, idx8b, acc, b0, b1, s0, s1):
    wid = _wid()
    base = wid * CH
    pltpu.sync_copy(nbq_img.at[wid], idx8a)
    pltpu.sync_copy(nbu_img.at[wid], idx8b)
    _gsum8(e1, idx8a, 0, CH, acc, b0, b1, s0, s1)
    pltpu.sync_copy(acc, out_ae1.at[pl.ds(base, CH)])
    _gsum8(e1u, idx8b, 0, CH, acc, b0, b1, s0, s1)
    pltpu.sync_copy(acc, out_ae1u.at[pl.ds(base, CH)])


_f32 = jnp.float32
_i32 = jnp.int32


@functools.cache
def _sc_mesh():
    return plsc.VectorSubcoreMesh(core_axis_name="c", subcore_axis_name="s")


def _run_sc_a(qt, ut, qn, qnb, unb, qst, embq, embq2, embu):
    out_type = [
        jax.ShapeDtypeStruct((N, 128), _i32),
        jax.ShapeDtypeStruct((N, 128), _i32),
        jax.ShapeDtypeStruct((N, 128), _i32),
        jax.ShapeDtypeStruct((N, DP), _f32),
        jax.ShapeDtypeStruct((N, DP), _f32),
        jax.ShapeDtypeStruct((N, DP), _f32),
        jax.ShapeDtypeStruct((N, DP), _f32),
    ]
    scratch = [
        pltpu.VMEM((CH,), _i32),
        pltpu.VMEM((CH, 128), _i32),
        pltpu.VMEM((CH, DP), _f32),
        pltpu.SemaphoreType.DMA,
    ]
    return pl.kernel(_sc_a, out_type=out_type, mesh=_sc_mesh(),
                     scratch_types=scratch)(qt, ut, qn, qnb, unb, qst,
                                            embq, embq2, embu)


def _run_sc_b(mu_img, ms_img, nbq_img, nbu_img, qsn_img,
              embq, embu, embs, embq2):
    out_type = [
        jax.ShapeDtypeStruct((NSP, DP), _f32),
        jax.ShapeDtypeStruct((NQP, DP), _f32),
        jax.ShapeDtypeStruct((N, DP), _f32),
        jax.ShapeDtypeStruct((N, DP), _f32),
        jax.ShapeDtypeStruct((SPQ * N, DP), _f32),
    ]
    scratch = [
        pltpu.VMEM((MU_CH // CH * NBR, 128), _i32),
        pltpu.VMEM((NBR, MS_CH), _i32),
        pltpu.VMEM((NBR, CH), _i32),
        pltpu.VMEM((NBR, CH), _i32),
        pltpu.VMEM((SPQ, CH), _i32),
        pltpu.VMEM((CH, DP), _f32),
        pltpu.VMEM((CH, DP), _f32),
        pltpu.VMEM((CH, DP), _f32),
        pltpu.SemaphoreType.DMA,
        pltpu.SemaphoreType.DMA,
    ]
    return pl.kernel(_sc_b, out_type=out_type, mesh=_sc_mesh(),
                     scratch_types=scratch)(mu_img, ms_img, nbq_img, nbu_img,
                                            qsn_img, embq, embu, embs, embq2)


def _run_sc_c(nbq_img, nbu_img, e1, e1u):
    out_type = [
        jax.ShapeDtypeStruct((N, DP), _f32),
        jax.ShapeDtypeStruct((N, DP), _f32),
    ]
    scratch = [
        pltpu.VMEM((NBR, CH), _i32),
        pltpu.VMEM((NBR, CH), _i32),
        pltpu.VMEM((CH, DP), _f32),
        pltpu.VMEM((CH, DP), _f32),
        pltpu.VMEM((CH, DP), _f32),
        pltpu.SemaphoreType.DMA,
        pltpu.SemaphoreType.DMA,
    ]
    return pl.kernel(_sc_c, out_type=out_type, mesh=_sc_mesh(),
                     scratch_types=scratch)(nbq_img, nbu_img, e1, e1u)


def _tc_e1(emb_ref, sum_ref, w_ref, b_ref, out_ref):
    x = emb_ref[...] + 0.125 * sum_ref[...]
    out_ref[...] = jnp.tanh(
        jnp.dot(x, w_ref[...], preferred_element_type=_f32) + b_ref[0:1, :])


def _run_tc_e1(emb_all, sum_all, w1p, b1p):
    rows = emb_all.shape[0]
    blk = 512
    return pl.pallas_call(
        _tc_e1,
        grid=(rows // blk,),
        in_specs=[
            pl.BlockSpec((blk, DP), lambda i: (i, 0)),
            pl.BlockSpec((blk, DP), lambda i: (i, 0)),
            pl.BlockSpec((DP, DP), lambda i: (0, 0)),
            pl.BlockSpec((8, DP), lambda i: (0, 0)),
        ],
        out_specs=pl.BlockSpec((blk, DP), lambda i: (i, 0)),
        out_shape=jax.ShapeDtypeStruct((rows, DP), _f32),
    )(emb_all, sum_all, w1p, b1p)


def _tc_rows(gq0, m1s, ae1s, gu0, mu1s, ae1us, gq2sel, mcol, rcol, embr,
             w0, b0, wl, bl, watt, batt, wih, bg, wk, bk, out_k):
    b0r = b0[0:1, :]
    blr = bl[0:1, :]

    x = gq0[...] + 0.125 * m1s[...]
    e0a = jnp.tanh(jnp.dot(x, w0[...], preferred_element_type=_f32) + b0r)
    x = e0a + 0.125 * ae1s[...]
    e0b = jnp.tanh(jnp.dot(x, w0[...], preferred_element_type=_f32) + b0r)
    aggq = jnp.tanh(jnp.dot(e0b, wl[...], preferred_element_type=_f32) + blr)

    x = gu0[...] + 0.125 * mu1s[...]
    e0a = jnp.tanh(jnp.dot(x, w0[...], preferred_element_type=_f32) + b0r)
    x = e0a + 0.125 * ae1us[...]
    e0b = jnp.tanh(jnp.dot(x, w0[...], preferred_element_type=_f32) + b0r)
    aggu = jnp.tanh(jnp.dot(e0b, wl[...], preferred_element_type=_f32) + blr)

    m = mcol[...] > 0.5
    eq1 = jnp.where(m, aggq, gq0[...])
    eq2 = jnp.where(m, aggu, gq2sel[...])
    eq = jnp.concatenate([eq1, eq2], axis=1)
    logits = jnp.dot(eq, watt[...], preferred_element_type=_f32) + batt[0:1, :]
    lmax = jnp.max(logits, axis=1, keepdims=True)
    ew = jnp.exp(logits - lmax)
    attn = ew / jnp.sum(ew, axis=1, keepdims=True)
    eq = eq * attn

    r = rcol[...]
    embr_row = embr[0:1, :] * (1.0 - r) + embr[1:2, :] * r
    x2 = jnp.concatenate([eq, embr_row], axis=1)
    gates = jnp.dot(x2, wih[...], preferred_element_type=_f32) + bg[0:1, :]
    gi = jax.nn.sigmoid(gates[:, 0:DP])
    gg = jnp.tanh(gates[:, 2 * DP:3 * DP])
    go = jax.nn.sigmoid(gates[:, 3 * DP:4 * DP])
    h = go * jnp.tanh(gi * gg)
    out_k[...] = jnp.dot(h, wk[...], preferred_element_type=_f32) + bk[0:1, :]


def _run_tc_rows(gq0, m1s, ae1s, gu0, mu1s, ae1us, gq2sel, mcol, rcol, embr,
                 w0, b0, wl, bl, watt, batt, wih, bg, wk, bk):
    blk = 512
    row_spec = pl.BlockSpec((blk, DP), lambda i: (i, 0))
    col_spec = pl.BlockSpec((blk, 1), lambda i: (i, 0))
    full = lambda a: pl.BlockSpec(a.shape, lambda i: tuple(0 for _ in a.shape))
    return pl.pallas_call(
        _tc_rows,
        grid=(N // blk,),
        in_specs=[row_spec] * 7 + [col_spec, col_spec] +
                 [full(a) for a in (embr, w0, b0, wl, bl, watt, batt, wih, bg,
                                    wk, bk)],
        out_specs=row_spec,
        out_shape=jax.ShapeDtypeStruct((N, DP), _f32),
    )(gq0, m1s, ae1s, gu0, mu1s, ae1us, gq2sel, mcol, rcol, embr,
      w0, b0, wl, bl, watt, batt, wih, bg, wk, bk)


def _tc_pred(qs_ref, k_ref, wq, bq, wwq, wwk, out_ref):
    qs = qs_ref[0]
    k = k_ref[0]
    q = jnp.dot(qs, wq[...], preferred_element_type=_f32) + bq[0:1, :]
    s = lax.dot_general(q, k, (((1,), (1,)), ((), ())),
                        preferred_element_type=_f32)
    qw = jnp.sum(q * wwq[0:1, :], axis=1, keepdims=True)
    kwf = lax.dot_general(wwk[...], k, (((1,), (1,)), ((), ())),
                          preferred_element_type=_f32)
    w = qw + kwf[0:1, :]
    ti = lax.broadcasted_iota(_i32, (S * 5, S), 0) // 5
    jj = lax.broadcasted_iota(_i32, (S * 5, S), 1)
    valid = (jj <= ti) & (ti <= S - 2)
    w = jnp.where(valid, w, -1e30)
    wmax = jnp.max(w)
    ew = jnp.where(valid, jnp.exp(w - wmax), 0.0)
    num = ew * jax.nn.sigmoid(s)
    seg = (lax.broadcasted_iota(_i32, (S, S * 5), 1) // 5
           == lax.broadcasted_iota(_i32, (S, S * 5), 0)).astype(_f32)
    ew_t = jnp.sum(jnp.dot(seg, ew, preferred_element_type=_f32),
                   axis=1, keepdims=True)
    num_t = jnp.sum(jnp.dot(seg, num, preferred_element_type=_f32),
                    axis=1, keepdims=True)
    out_ref[...] = (num_t / (ew_t + 1e-30)).reshape(1, S, 1)


def _run_tc_pred(qs_all, k_all, wqp, bqp, wwq, wwk):
    return pl.pallas_call(
        _tc_pred,
        grid=(B,),
        in_specs=[
            pl.BlockSpec((1, S * 5, DP), lambda b: (b, 0, 0)),
            pl.BlockSpec((1, S, DP), lambda b: (b, 0, 0)),
            pl.BlockSpec((DP, DP), lambda b: (0, 0)),
            pl.BlockSpec((8, DP), lambda b: (0, 0)),
            pl.BlockSpec((8, DP), lambda b: (0, 0)),
            pl.BlockSpec((8, DP), lambda b: (0, 0)),
        ],
        out_specs=pl.BlockSpec((1, S, 1), lambda b: (b, 0, 0)),
        out_shape=jax.ShapeDtypeStruct((B, S, 1), _f32),
    )(qs_all, k_all, wqp, bqp, wwq, wwk)


def _pad_rows_cols(a, rows, cols):
    return jnp.pad(a, ((0, rows - a.shape[0]), (0, cols - a.shape[1])))


def _pad_vec_row(v, cols, fill=0.0):
    vp = jnp.pad(v.astype(_f32), (0, cols - v.shape[0]),
                 constant_values=fill)
    return jnp.broadcast_to(vp[None, :], (8, cols))


def _pad_block_matrix(w, in_blocks, out_blocks, blk_in=D, blk_out=D):
    out = jnp.zeros((in_blocks * DP, out_blocks * DP), _f32)
    for i in range(in_blocks):
        for j in range(out_blocks):
            out = out.at[i * DP:i * DP + blk_in, j * DP:j * DP + blk_out].set(
                w[i * blk_in:(i + 1) * blk_in, j * blk_out:(j + 1) * blk_out])
    return out


def _pad_block_vec(v, blocks, fill=0.0):
    out = jnp.full((blocks * DP,), fill, _f32)
    for i in range(blocks):
        out = out.at[i * DP:i * DP + D].set(v[i * D:(i + 1) * D])
    return jnp.broadcast_to(out[None, :], (8, blocks * DP))


def kernel(user, question, response, mask, q_neighbors, s_neighbors,
           u_neighbors, q_neighbors_2, qs_table, emb_q, emb_s, emb_u,
           emb_q2, emb_r, W_ih, W_hh, b_ih, b_hh, agg_W, agg_b,
           W_agg_last, b_agg_last, W_att, b_att, W_query, b_query,
           W_key, b_key, W_w, b_w):
    i32 = lambda a: a.astype(_i32)
    qt = i32(question.reshape(-1))
    ut = i32(user.reshape(-1))
    qn = i32(jnp.concatenate([question[:, 1:], question[:, -1:]],
                             axis=1).reshape(-1))

    embq_p = jnp.pad(emb_q, ((0, 0), (0, DP - D)))
    embs_p = jnp.pad(emb_s, ((0, 0), (0, DP - D)))
    embu_p = jnp.pad(emb_u, ((0, 0), (0, DP - D)))
    embq2_p = jnp.pad(emb_q2, ((0, 0), (0, DP - D)))
    qnb_p = jnp.pad(i32(q_neighbors), ((0, 0), (0, 128 - NBR)))
    unb_p = jnp.pad(i32(u_neighbors), ((0, 0), (0, 128 - NBR)))
    qst_p = jnp.pad(i32(qs_table), ((0, 0), (0, 128 - SPQ)))
    mu_img = (jnp.pad(i32(q_neighbors_2), ((0, NQP - NQ), (0, 0)))
              .reshape(NW, MU_CH // CH, CH, NBR).transpose(0, 1, 3, 2)
              .reshape(NW, MU_CH // CH * NBR, CH))
    ms_img = (jnp.pad(i32(s_neighbors), ((0, NSP - NS), (0, 0)))
              .reshape(NW, MS_CH, NBR).transpose(0, 2, 1))

    nbq, nbu, qsn, g_q0, g_q2sel, g_u0, g_qnext = _run_sc_a(
        qt, ut, qn, qnb_p, unb_p, qst_p, embq_p, embq2_p, embu_p)
    nbq_img = nbq[:, :NBR].reshape(NW, CH, NBR).transpose(0, 2, 1)
    nbu_img = nbu[:, :NBR].reshape(NW, CH, NBR).transpose(0, 2, 1)
    qsn_img = qsn[:, :SPQ].reshape(NW, CH, SPQ).transpose(0, 2, 1)

    ms_sum, mu_sum, m1_sum, mu1_sum, skl = _run_sc_b(
        mu_img, ms_img, nbq_img, nbu_img, qsn_img,
        embq_p, embu_p, embs_p, embq2_p)

    w1p = _pad_rows_cols(agg_W[1], DP, DP)
    b1p = _pad_vec_row(agg_b[1], DP)
    emb_tab = jnp.concatenate(
        [jnp.pad(embs_p, ((0, NSP - NS), (0, 0))),
         jnp.pad(embq2_p, ((0, NQP - NQ), (0, 0)))], axis=0)
    sum_tab = jnp.concatenate([ms_sum, mu_sum], axis=0)
    e_all = _run_tc_e1(emb_tab, sum_tab, w1p, b1p)
    e1 = e_all[:NSP]
    e1u = e_all[NSP:]

    ae1_sum, ae1u_sum = _run_sc_c(nbq_img, nbu_img, e1, e1u)

    w0p = _pad_rows_cols(agg_W[0], DP, DP)
    b0p = _pad_vec_row(agg_b[0], DP)
    wlp = _pad_rows_cols(W_agg_last, DP, DP)
    blp = _pad_vec_row(b_agg_last, DP)
    wattp = _pad_block_matrix(W_att, 2, 2)
    battp = _pad_block_vec(b_att, 2, fill=-1e30)
    wihp = _pad_block_matrix(W_ih.T, 3, 4)
    bgp = _pad_block_vec(b_ih + b_hh, 4)
    wkp = _pad_rows_cols(W_key, DP, DP)
    bkp = _pad_vec_row(b_key, DP)
    embr_p = jnp.pad(emb_r, ((0, 6), (0, DP - D)))
    mcol = (mask.reshape(-1, 1)).astype(_f32)
    rcol = (response.reshape(-1, 1)).astype(_f32)
    k_all = _run_tc_rows(g_q0, m1_sum, ae1_sum, g_u0, mu1_sum, ae1u_sum,
                         g_q2sel, mcol, rcol, embr_p,
                         w0p, b0p, wlp, blp, wattp, battp, wihp, bgp,
                         wkp, bkp)

    qs_stack = jnp.stack(
        [g_qnext] + [skl[j * N:(j + 1) * N] for j in range(SPQ)],
        axis=1)
    qs_all = qs_stack.reshape(B, S * 5, DP)
    k_resh = k_all.reshape(B, S, DP)
    wqp = _pad_rows_cols(W_query, DP, DP)
    bqp = _pad_vec_row(b_query, DP)
    wwq = _pad_vec_row(W_w[:D, 0], DP)
    wwk = _pad_vec_row(W_w[D:, 0], DP)
    out_T = _run_tc_pred(qs_all, k_resh, wqp, bqp, wwq, wwk)

    y = out_T[:, :, 0]
    return jnp.concatenate([jnp.full((B, 1), 0.5, _f32), y[:, :S - 1]],
                           axis=1)


# speedup vs baseline: 7.3498x; 1.0787x over previous
import functools
import jax
import jax.numpy as jnp
from jax import lax
from jax.experimental import pallas as pl
from jax.experimental.pallas import tpu as pltpu
from jax.experimental.pallas import tpu_sc as plsc

NQ, NS, NU = 20000, 1000, 5000
D = 100
DP = 128
NBR = 8
SPQ = 4
B, S = 64, 64
N = B * S
NSP = 1024
NQP = 20480

NC, NSC = 2, 16
NW = NC * NSC
CH = N // NW
MS_CH = NSP // NW
MU_CH = NQP // NW


def _wid():
    return lax.axis_index("s") * NC + lax.axis_index("c")


def _acc(acc, buf, nrows):
    def body(r, _):
        for c in range(7):
            sl = pl.ds(c * 16, 16)
            plsc.addupdate(acc.at[r, sl], buf[r, sl])
        return 0
    lax.fori_loop(0, nrows, body, 0, unroll=2)


def _acci(acc, buf, s):
    def body(m, _):
        r8 = m * 8
        for c in range(7):
            sl = pl.ds(c * 16, 16)
            v = ((buf[r8, sl] + buf[r8 + 1, sl]) +
                 (buf[r8 + 2, sl] + buf[r8 + 3, sl])) + \
                ((buf[r8 + 4, sl] + buf[r8 + 5, sl]) +
                 (buf[r8 + 6, sl] + buf[r8 + 7, sl]))
            acc[s * 16 + m, sl] = v
        return 0
    lax.fori_loop(0, 16, body, 0)


def _gsum8i(tab, flat, off, nsub, acc, b0, b1, s0, s1):
    def idx(s):
        return flat.at[pl.ds(off + s * 128, 128)]
    cps = [None] * nsub
    cps[0] = pltpu.async_copy(tab.at[idx(0)], b0, s0)
    if nsub > 1:
        cps[1] = pltpu.async_copy(tab.at[idx(1)], b1, s1)
    for s in range(nsub):
        cps[s].wait()
        _acci(acc, b0 if s % 2 == 0 else b1, s)
        if s + 2 < nsub:
            cps[s + 2] = pltpu.async_copy(
                tab.at[idx(s + 2)],
                b0 if (s + 2) % 2 == 0 else b1,
                s0 if (s + 2) % 2 == 0 else s1)


def _gsum8(tab, idx2d, joff, nrows, acc, b0, b1, s0, s1):
    dst_a = acc if nrows == CH else acc.at[pl.ds(0, nrows)]
    dst0 = b0 if nrows == CH else b0.at[pl.ds(0, nrows)]
    dst1 = b1 if nrows == CH else b1.at[pl.ds(0, nrows)]
    cps = [None] * NBR
    cps[0] = pltpu.async_copy(tab.at[idx2d.at[joff]], dst_a, s0)
    cps[1] = pltpu.async_copy(tab.at[idx2d.at[joff + 1]], dst1, s1)
    cps[0].wait()
    for j in range(2, NBR + 1):
        if j < NBR:
            cps[j] = pltpu.async_copy(
                tab.at[idx2d.at[joff + j]],
                dst0 if j % 2 == 0 else dst1,
                s0 if j % 2 == 0 else s1)
        cps[j - 1].wait()
        _acc(acc, b0 if (j - 1) % 2 == 0 else b1, nrows)


def _sc_a(qt, ut, qn, qnb, unb, qst, embq, embq2, embu,
          out_nbq, out_nbu, out_qsn, out_gq0, out_gq2sel, out_gu0, out_gqn,
          idx_v, ibuf, fbuf, sem):
    base = _wid() * CH
    sl = pl.ds(base, CH)
    pltpu.sync_copy(qt.at[sl], idx_v)
    pltpu.async_copy(qnb.at[idx_v], ibuf, sem).wait()
    pltpu.sync_copy(ibuf, out_nbq.at[sl])
    pltpu.async_copy(embq.at[idx_v], fbuf, sem).wait()
    pltpu.sync_copy(fbuf, out_gq0.at[sl])
    pltpu.async_copy(embq2.at[idx_v], fbuf, sem).wait()
    pltpu.sync_copy(fbuf, out_gq2sel.at[sl])
    pltpu.sync_copy(ut.at[sl], idx_v)
    pltpu.async_copy(unb.at[idx_v], ibuf, sem).wait()
    pltpu.sync_copy(ibuf, out_nbu.at[sl])
    pltpu.async_copy(embu.at[idx_v], fbuf, sem).wait()
    pltpu.sync_copy(fbuf, out_gu0.at[sl])
    pltpu.sync_copy(qn.at[sl], idx_v)
    pltpu.async_copy(qst.at[idx_v], ibuf, sem).wait()
    pltpu.sync_copy(ibuf, out_qsn.at[sl])
    pltpu.async_copy(embq.at[idx_v], fbuf, sem).wait()
    pltpu.sync_copy(fbuf, out_gqn.at[sl])


def _sc_b(qn2f, snbf, nbq_img, nbu_img, qsn_img, embq, embu, embs, embq2,
          out_ms, out_mu, out_m1, out_mu1, out_skl,
          raw_mu, raw_ms, idx8a, idx8b, idx4, acc, b0, b1, s0, s1):
    wid = _wid()
    pltpu.sync_copy(qn2f.at[pl.ds(wid * MU_CH * NBR, MU_CH * NBR)], raw_mu)
    pltpu.sync_copy(snbf.at[pl.ds(wid * MS_CH * NBR, MS_CH * NBR)], raw_ms)
    pltpu.sync_copy(nbq_img.at[wid], idx8a)
    pltpu.sync_copy(nbu_img.at[wid], idx8b)
    pltpu.sync_copy(qsn_img.at[wid], idx4)

    z16 = jnp.zeros((16,), _f32)
    def zbody(r, _):
        acc[r, pl.ds(112, 16)] = z16
        return 0
    lax.fori_loop(0, CH, zbody, 0, unroll=4)

    sbase = wid * MS_CH
    _gsum8i(embq, raw_ms, 0, MS_CH // 16, acc, b0, b1, s0, s1)
    pltpu.sync_copy(acc.at[pl.ds(0, MS_CH)], out_ms.at[pl.ds(sbase, MS_CH)])

    def mu_chunk(c, _):
        _gsum8i(embu, raw_mu, c * (CH * NBR), CH // 16, acc, b0, b1, s0, s1)
        pltpu.sync_copy(acc, out_mu.at[pl.ds(wid * MU_CH + c * CH, CH)])
        return 0
    lax.fori_loop(0, MU_CH // CH, mu_chunk, 0)

    base = wid * CH
    _gsum8(embs, idx8a, 0, CH, acc, b0, b1, s0, s1)
    pltpu.sync_copy(acc, out_m1.at[pl.ds(base, CH)])

    _gsum8(embq2, idx8b, 0, CH, acc, b0, b1, s0, s1)
    pltpu.sync_copy(acc, out_mu1.at[pl.ds(base, CH)])

    cps = [None] * SPQ
    cps[0] = pltpu.async_copy(embs.at[idx4.at[0]], b0, s0)
    cps[1] = pltpu.async_copy(embs.at[idx4.at[1]], b1, s1)
    for j in range(SPQ):
        cps[j].wait()
        pltpu.sync_copy(b0 if j % 2 == 0 else b1,
                        out_skl.at[pl.ds(j * N + base, CH)])
        if j + 2 < SPQ:
            cps[j + 2] = pltpu.async_copy(
                embs.at[idx4.at[j + 2]],
                b0 if (j + 2) % 2 == 0 else b1,
                s0 if (j + 2) % 2 == 0 else s1)


def _sc_c(nbq_img, nbu_img, e1, e1u, out_ae1, out_ae1u,
          idx8a, idx8b, acc, b0, b1, s0, s1):
    wid = _wid()
    base = wid * CH
    pltpu.sync_copy(nbq_img.at[wid], idx8a)
    pltpu.sync_copy(nbu_img.at[wid], idx8b)
    _gsum8(e1, idx8a, 0, CH, acc, b0, b1, s0, s1)
    pltpu.sync_copy(acc, out_ae1.at[pl.ds(base, CH)])
    _gsum8(e1u, idx8b, 0, CH, acc, b0, b1, s0, s1)
    pltpu.sync_copy(acc, out_ae1u.at[pl.ds(base, CH)])


_f32 = jnp.float32
_i32 = jnp.int32


@functools.cache
def _sc_mesh():
    return plsc.VectorSubcoreMesh(core_axis_name="c", subcore_axis_name="s")


def _run_sc_a(qt, ut, qn, qnb, unb, qst, embq, embq2, embu):
    out_type = [
        jax.ShapeDtypeStruct((N, 128), _i32),
        jax.ShapeDtypeStruct((N, 128), _i32),
        jax.ShapeDtypeStruct((N, 128), _i32),
        jax.ShapeDtypeStruct((N, DP), _f32),
        jax.ShapeDtypeStruct((N, DP), _f32),
        jax.ShapeDtypeStruct((N, DP), _f32),
        jax.ShapeDtypeStruct((N, DP), _f32),
    ]
    scratch = [
        pltpu.VMEM((CH,), _i32),
        pltpu.VMEM((CH, 128), _i32),
        pltpu.VMEM((CH, DP), _f32),
        pltpu.SemaphoreType.DMA,
    ]
    return pl.kernel(_sc_a, out_type=out_type, mesh=_sc_mesh(),
                     scratch_types=scratch)(qt, ut, qn, qnb, unb, qst,
                                            embq, embq2, embu)


def _run_sc_b(qn2f, snbf, nbq_img, nbu_img, qsn_img,
              embq, embu, embs, embq2):
    out_type = [
        jax.ShapeDtypeStruct((NSP, DP), _f32),
        jax.ShapeDtypeStruct((NQP, DP), _f32),
        jax.ShapeDtypeStruct((N, DP), _f32),
        jax.ShapeDtypeStruct((N, DP), _f32),
        jax.ShapeDtypeStruct((SPQ * N, DP), _f32),
    ]
    scratch = [
        pltpu.VMEM((MU_CH * NBR,), _i32),
        pltpu.VMEM((MS_CH * NBR,), _i32),
        pltpu.VMEM((NBR, CH), _i32),
        pltpu.VMEM((NBR, CH), _i32),
        pltpu.VMEM((SPQ, CH), _i32),
        pltpu.VMEM((CH, DP), _f32),
        pltpu.VMEM((CH, DP), _f32),
        pltpu.VMEM((CH, DP), _f32),
        pltpu.SemaphoreType.DMA,
        pltpu.SemaphoreType.DMA,
    ]
    return pl.kernel(_sc_b, out_type=out_type, mesh=_sc_mesh(),
                     scratch_types=scratch)(qn2f, snbf, nbq_img, nbu_img,
                                            qsn_img, embq, embu, embs, embq2)


def _run_sc_c(nbq_img, nbu_img, e1, e1u):
    out_type = [
        jax.ShapeDtypeStruct((N, DP), _f32),
        jax.ShapeDtypeStruct((N, DP), _f32),
    ]
    scratch = [
        pltpu.VMEM((NBR, CH), _i32),
        pltpu.VMEM((NBR, CH), _i32),
        pltpu.VMEM((CH, DP), _f32),
        pltpu.VMEM((CH, DP), _f32),
        pltpu.VMEM((CH, DP), _f32),
        pltpu.SemaphoreType.DMA,
        pltpu.SemaphoreType.DMA,
    ]
    return pl.kernel(_sc_c, out_type=out_type, mesh=_sc_mesh(),
                     scratch_types=scratch)(nbq_img, nbu_img, e1, e1u)


def _tc_e1(emb_ref, sum_ref, w_ref, b_ref, out_ref):
    x = emb_ref[...] + 0.125 * sum_ref[...]
    out_ref[...] = jnp.tanh(
        jnp.dot(x, w_ref[...], preferred_element_type=_f32) + b_ref[0:1, :])


def _run_tc_e1(emb_tab, sum_tab, w1p, b1p):
    rows = emb_tab.shape[0]
    blk = 512
    return pl.pallas_call(
        _tc_e1,
        grid=(pl.cdiv(rows, blk),),
        in_specs=[
            pl.BlockSpec((blk, DP), lambda i: (i, 0)),
            pl.BlockSpec((blk, DP), lambda i: (i, 0)),
            pl.BlockSpec((DP, DP), lambda i: (0, 0)),
            pl.BlockSpec((8, DP), lambda i: (0, 0)),
        ],
        out_specs=pl.BlockSpec((blk, DP), lambda i: (i, 0)),
        out_shape=jax.ShapeDtypeStruct((rows, DP), _f32),
    )(emb_tab, sum_tab, w1p, b1p)


def _tc_rows(gq0, m1s, ae1s, gu0, mu1s, ae1us, gq2sel, mcol, rcol, embr,
             w0, b0, wl, bl, watt, batt, wih, bg, wk, bk, out_k):
    b0r = b0[0:1, :]
    blr = bl[0:1, :]

    x = gq0[...] + 0.125 * m1s[...]
    e0a = jnp.tanh(jnp.dot(x, w0[...], preferred_element_type=_f32) + b0r)
    x = e0a + 0.125 * ae1s[...]
    e0b = jnp.tanh(jnp.dot(x, w0[...], preferred_element_type=_f32) + b0r)
    aggq = jnp.tanh(jnp.dot(e0b, wl[...], preferred_element_type=_f32) + blr)

    x = gu0[...] + 0.125 * mu1s[...]
    e0a = jnp.tanh(jnp.dot(x, w0[...], preferred_element_type=_f32) + b0r)
    x = e0a + 0.125 * ae1us[...]
    e0b = jnp.tanh(jnp.dot(x, w0[...], preferred_element_type=_f32) + b0r)
    aggu = jnp.tanh(jnp.dot(e0b, wl[...], preferred_element_type=_f32) + blr)

    m = mcol[...] > 0.5
    eq1 = jnp.where(m, aggq, gq0[...])
    eq2 = jnp.where(m, aggu, gq2sel[...])
    eq = jnp.concatenate([eq1, eq2], axis=1)
    logits = jnp.dot(eq, watt[...], preferred_element_type=_f32) + batt[0:1, :]
    lmax = jnp.max(logits, axis=1, keepdims=True)
    ew = jnp.exp(logits - lmax)
    attn = ew / jnp.sum(ew, axis=1, keepdims=True)
    eq = eq * attn

    r = rcol[...]
    embr_row = embr[0:1, :] * (1.0 - r) + embr[1:2, :] * r
    x2 = jnp.concatenate([eq, embr_row], axis=1)
    gates = jnp.dot(x2, wih[...], preferred_element_type=_f32) + bg[0:1, :]
    gi = jax.nn.sigmoid(gates[:, 0:DP])
    gg = jnp.tanh(gates[:, 2 * DP:3 * DP])
    go = jax.nn.sigmoid(gates[:, 3 * DP:4 * DP])
    h = go * jnp.tanh(gi * gg)
    out_k[...] = jnp.dot(h, wk[...], preferred_element_type=_f32) + bk[0:1, :]


def _run_tc_rows(gq0, m1s, ae1s, gu0, mu1s, ae1us, gq2sel, mcol, rcol, embr,
                 w0, b0, wl, bl, watt, batt, wih, bg, wk, bk):
    blk = 512
    row_spec = pl.BlockSpec((blk, DP), lambda i: (i, 0))
    col_spec = pl.BlockSpec((blk, 1), lambda i: (i, 0))
    full = lambda a: pl.BlockSpec(a.shape, lambda i: tuple(0 for _ in a.shape))
    return pl.pallas_call(
        _tc_rows,
        grid=(N // blk,),
        in_specs=[row_spec] * 7 + [col_spec, col_spec] +
                 [full(a) for a in (embr, w0, b0, wl, bl, watt, batt, wih, bg,
                                    wk, bk)],
        out_specs=row_spec,
        out_shape=jax.ShapeDtypeStruct((N, DP), _f32),
    )(gq0, m1s, ae1s, gu0, mu1s, ae1us, gq2sel, mcol, rcol, embr,
      w0, b0, wl, bl, watt, batt, wih, bg, wk, bk)


def _tc_pred(qn_ref, skl_ref, k_ref, wq, bq, wwq, wwk, out_ref):
    k = k_ref[0]
    kwf = lax.dot_general(wwk[...], k, (((1,), (1,)), ((), ())),
                          preferred_element_type=_f32)
    krow = kwf[0:1, :]
    ti = lax.broadcasted_iota(_i32, (S, S), 0)
    jj = lax.broadcasted_iota(_i32, (S, S), 1)
    valid = (jj <= ti) & (ti <= S - 2)
    qs = [qn_ref[0]] + [skl_ref[j, 0] for j in range(SPQ)]
    qps, ws = [], []
    for x in qs:
        q = jnp.dot(x, wq[...], preferred_element_type=_f32) + bq[0:1, :]
        qps.append(q)
        w = jnp.sum(q * wwq[0:1, :], axis=1, keepdims=True) + krow
        ws.append(jnp.where(valid, w, -1e30))
    wmax = jnp.max(ws[0])
    for w in ws[1:]:
        wmax = jnp.maximum(wmax, jnp.max(w))
    num = jnp.zeros((S, 1), _f32)
    den = jnp.zeros((S, 1), _f32)
    for q, w in zip(qps, ws):
        ew = jnp.where(valid, jnp.exp(w - wmax), 0.0)
        s = lax.dot_general(q, k, (((1,), (1,)), ((), ())),
                            preferred_element_type=_f32)
        num = num + jnp.sum(ew * jax.nn.sigmoid(s), axis=1, keepdims=True)
        den = den + jnp.sum(ew, axis=1, keepdims=True)
    out_ref[...] = (num / (den + 1e-30)).reshape(1, S, 1)


def _run_tc_pred(qn_all, skl_all, k_all, wqp, bqp, wwq, wwk):
    return pl.pallas_call(
        _tc_pred,
        grid=(B,),
        in_specs=[
            pl.BlockSpec((1, S, DP), lambda b: (b, 0, 0)),
            pl.BlockSpec((SPQ, 1, S, DP), lambda b: (0, b, 0, 0)),
            pl.BlockSpec((1, S, DP), lambda b: (b, 0, 0)),
            pl.BlockSpec((DP, DP), lambda b: (0, 0)),
            pl.BlockSpec((8, DP), lambda b: (0, 0)),
            pl.BlockSpec((8, DP), lambda b: (0, 0)),
            pl.BlockSpec((8, DP), lambda b: (0, 0)),
        ],
        out_specs=pl.BlockSpec((1, S, 1), lambda b: (b, 0, 0)),
        out_shape=jax.ShapeDtypeStruct((B, S, 1), _f32),
    )(qn_all, skl_all, k_all, wqp, bqp, wwq, wwk)


def _pad_rows_cols(a, rows, cols):
    return jnp.pad(a, ((0, rows - a.shape[0]), (0, cols - a.shape[1])))


def _pad_vec_row(v, cols, fill=0.0):
    vp = jnp.pad(v.astype(_f32), (0, cols - v.shape[0]),
                 constant_values=fill)
    return jnp.broadcast_to(vp[None, :], (8, cols))


def _pad_block_matrix(w, in_blocks, out_blocks, blk_in=D, blk_out=D):
    out = jnp.zeros((in_blocks * DP, out_blocks * DP), _f32)
    for i in range(in_blocks):
        for j in range(out_blocks):
            out = out.at[i * DP:i * DP + blk_in, j * DP:j * DP + blk_out].set(
                w[i * blk_in:(i + 1) * blk_in, j * blk_out:(j + 1) * blk_out])
    return out


def _pad_block_vec(v, blocks, fill=0.0):
    out = jnp.full((blocks * DP,), fill, _f32)
    for i in range(blocks):
        out = out.at[i * DP:i * DP + D].set(v[i * D:(i + 1) * D])
    return jnp.broadcast_to(out[None, :], (8, blocks * DP))


def kernel(user, question, response, mask, q_neighbors, s_neighbors,
           u_neighbors, q_neighbors_2, qs_table, emb_q, emb_s, emb_u,
           emb_q2, emb_r, W_ih, W_hh, b_ih, b_hh, agg_W, agg_b,
           W_agg_last, b_agg_last, W_att, b_att, W_query, b_query,
           W_key, b_key, W_w, b_w):
    i32 = lambda a: a.astype(_i32)
    qt = i32(question.reshape(-1))
    ut = i32(user.reshape(-1))
    qn = i32(jnp.concatenate([question[:, 1:], question[:, -1:]],
                             axis=1).reshape(-1))

    embq_p = jnp.pad(emb_q, ((0, 0), (0, DP - D)))
    embs_p = jnp.pad(emb_s, ((0, 0), (0, DP - D)))
    embu_p = jnp.pad(emb_u, ((0, 0), (0, DP - D)))
    embq2_p = jnp.pad(emb_q2, ((0, 0), (0, DP - D)))
    qnb_p = jnp.pad(i32(q_neighbors), ((0, 0), (0, 128 - NBR)))
    unb_p = jnp.pad(i32(u_neighbors), ((0, 0), (0, 128 - NBR)))
    qst_p = jnp.pad(i32(qs_table), ((0, 0), (0, 128 - SPQ)))
    qn2f = jnp.pad(i32(q_neighbors_2), ((0, NQP - NQ), (0, 0))).reshape(-1)
    snbf = jnp.pad(i32(s_neighbors), ((0, NSP - NS), (0, 0))).reshape(-1)

    nbq, nbu, qsn, g_q0, g_q2sel, g_u0, g_qnext = _run_sc_a(
        qt, ut, qn, qnb_p, unb_p, qst_p, embq_p, embq2_p, embu_p)
    nbq_img = nbq[:, :NBR].reshape(NW, CH, NBR).transpose(0, 2, 1)
    nbu_img = nbu[:, :NBR].reshape(NW, CH, NBR).transpose(0, 2, 1)
    qsn_img = qsn[:, :SPQ].reshape(NW, CH, SPQ).transpose(0, 2, 1)

    ms_sum, mu_sum, m1_sum, mu1_sum, skl = _run_sc_b(
        qn2f, snbf, nbq_img, nbu_img, qsn_img,
        embq_p, embu_p, embs_p, embq2_p)

    w1p = _pad_rows_cols(agg_W[1], DP, DP)
    b1p = _pad_vec_row(agg_b[1], DP)
    e1 = _run_tc_e1(embs_p, ms_sum, w1p, b1p)
    e1u = _run_tc_e1(embq2_p, mu_sum, w1p, b1p)

    ae1_sum, ae1u_sum = _run_sc_c(nbq_img, nbu_img, e1, e1u)

    w0p = _pad_rows_cols(agg_W[0], DP, DP)
    b0p = _pad_vec_row(agg_b[0], DP)
    wlp = _pad_rows_cols(W_agg_last, DP, DP)
    blp = _pad_vec_row(b_agg_last, DP)
    wattp = _pad_block_matrix(W_att, 2, 2)
    battp = _pad_block_vec(b_att, 2, fill=-1e30)
    wihp = _pad_block_matrix(W_ih.T, 3, 4)
    bgp = _pad_block_vec(b_ih + b_hh, 4)
    wkp = _pad_rows_cols(W_key, DP, DP)
    bkp = _pad_vec_row(b_key, DP)
    embr_p = jnp.pad(emb_r, ((0, 6), (0, DP - D)))
    mcol = (mask.reshape(-1, 1)).astype(_f32)
    rcol = (response.reshape(-1, 1)).astype(_f32)
    k_all = _run_tc_rows(g_q0, m1_sum, ae1_sum, g_u0, mu1_sum, ae1u_sum,
                         g_q2sel, mcol, rcol, embr_p,
                         w0p, b0p, wlp, blp, wattp, battp, wihp, bgp,
                         wkp, bkp)

    qn_all = g_qnext.reshape(B, S, DP)
    skl_all = skl.reshape(SPQ, B, S, DP)
    k_resh = k_all.reshape(B, S, DP)
    wqp = _pad_rows_cols(W_query, DP, DP)
    bqp = _pad_vec_row(b_query, DP)
    wwq = _pad_vec_row(W_w[:D, 0], DP)
    wwk = _pad_vec_row(W_w[D:, 0], DP)
    out_T = _run_tc_pred(qn_all, skl_all, k_resh, wqp, bqp, wwq, wwk)

    y = out_T[:, :, 0]
    return jnp.concatenate([jnp.full((B, 1), 0.5, _f32), y[:, :S - 1]],
                           axis=1)
